# baseline (speedup 1.0000x reference)
import functools

import jax
import jax.numpy as jnp
from jax import lax
from jax.experimental import pallas as pl
from jax.experimental.pallas import tpu as pltpu
from jax.experimental.pallas import tpu_sc as plsc

B = 4096
L = 200
EMB = 64
NW = 32
BPW = B // NW
CHUNKS = ((0, 128), (72, 128))
QV = EMB // 16
WPAIR = 2 * EMB
DICT_PAIRS = 500000


def _body(idx_hbm, off_hbm, w_hbm, out_hbm, idx_v, off_v, rows_v, out_v,
          sem0, sem1):
    c = lax.axis_index("c")
    s = lax.axis_index("s")
    wid = s * 2 + c
    base = wid * BPW * L

    pltpu.sync_copy(idx_hbm.at[pl.ds(wid * BPW * 2, BPW * 2)], idx_v)
    pltpu.sync_copy(off_hbm.at[pl.ds(base, BPW * L)], off_v)

    sems = (sem0, sem1)

    def start(b, slot):
        for j, (o, n) in enumerate(CHUNKS):
            pltpu.async_copy(
                w_hbm.at[idx_v.at[b * 2 + j]],
                rows_v.at[slot, pl.ds(o, n)],
                sems[slot],
            )

    def wait(slot):
        for j, (o, n) in enumerate(CHUNKS):
            pltpu.make_async_copy(
                w_hbm.at[idx_v.at[j]],
                rows_v.at[slot, pl.ds(o, n)],
                sems[slot],
            ).wait()

    start(0, 0)
    start(1, 1)

    def accum(slot, b):
        def step(acc, start_l, offs, lanes):
            acc = list(acc)
            for i in lanes:
                half = offs[i]
                l = start_l + i
                for q in range(QV):
                    acc[q] = acc[q] + rows_v[slot, l, pl.ds(half + 16 * q, 16)]
            return tuple(acc)

        def inner(g, acc):
            start_l = g * 16
            offs = off_v[pl.ds(b * L + start_l, 16)]
            return step(acc, start_l, offs, range(16))

        zero = jnp.zeros((16,), jnp.float32)
        acc = lax.fori_loop(0, L // 16, inner, (zero,) * QV)
        offs_t = off_v[pl.ds(b * L + (L - 16), 16)]
        acc = step(acc, L - 16, offs_t, range(8, 16))
        scale = jnp.float32(1.0 / L)
        for q in range(QV):
            out_v[b // 2, pl.ds((b % 2) * EMB + 16 * q, 16)] = acc[q] * scale

    def outer(g, carry):
        for slot in range(2):
            b = g * 2 + slot
            wait(slot)
            accum(slot, b)
            nb = b + 2

            @pl.when(nb < BPW)
            def _():
                start(nb, slot)
        return carry

    lax.fori_loop(0, BPW // 2, outer, 0)

    pltpu.sync_copy(out_v, out_hbm.at[pl.ds(wid * (BPW // 2), BPW // 2)])


_mesh = plsc.VectorSubcoreMesh(core_axis_name="c", subcore_axis_name="s")

_sc_call = pl.kernel(
    _body,
    mesh=_mesh,
    out_type=jax.ShapeDtypeStruct((B // 2, WPAIR), jnp.float32),
    scratch_types=[
        pltpu.VMEM((BPW * 2, 128), jnp.int32),
        pltpu.VMEM((BPW * L,), jnp.int32),
        pltpu.VMEM((2, L, WPAIR), jnp.float32),
        pltpu.VMEM((BPW // 2, WPAIR), jnp.float32),
        pltpu.SemaphoreType.DMA,
        pltpu.SemaphoreType.DMA,
    ],
    compiler_params=pltpu.CompilerParams(use_tc_tiling_on_sc=True),
)


@jax.jit
def _run(x, w):
    idx2 = x >> 1
    idx2c = jnp.stack([idx2[:, 0:128], idx2[:, L - 128:L]], axis=1)
    idx2c = idx2c.reshape(B * 2, 128)
    off = ((x & 1) * EMB).reshape(B * L)
    w2 = w.reshape(DICT_PAIRS, WPAIR)
    out2 = _sc_call(idx2c, off, w2)
    return out2.reshape(B, EMB)


def kernel(x, sizes, emb_weight):
    del sizes
    return _run(x, emb_weight)


# speedup vs baseline: 1.5085x; 1.5085x over previous
import functools

import jax
import jax.numpy as jnp
from jax import lax
from jax.experimental import pallas as pl
from jax.experimental.pallas import tpu as pltpu
from jax.experimental.pallas import tpu_sc as plsc

B = 4096
L = 200
EMB = 64
NW = 32
BPW = B // NW
CHUNKS = ((0, 128), (128, 72))
QV = EMB // 16
WPAD = 2 * EMB


def _body(idx_hbm, w_hbm, out_hbm, idx_v, rows_v, out_v, sem0, sem1):
    c = lax.axis_index("c")
    s = lax.axis_index("s")
    wid = s * 2 + c
    base = wid * BPW * L

    pltpu.sync_copy(idx_hbm.at[pl.ds(base, BPW * L)], idx_v)

    sems = (sem0, sem1)

    def start(b, slot):
        for (o, n) in CHUNKS:
            pltpu.async_copy(
                w_hbm.at[idx_v.at[pl.ds(b * L + o, n)]],
                rows_v.at[slot, pl.ds(o, n)],
                sems[slot],
            )

    def wait(slot):
        for (o, n) in CHUNKS:
            pltpu.make_async_copy(
                w_hbm.at[idx_v.at[pl.ds(o, n)]],
                rows_v.at[slot, pl.ds(o, n)],
                sems[slot],
            ).wait()

    start(0, 0)
    start(1, 1)

    def accum(slot, b):
        def inner(l, acc):
            return tuple(
                acc[q] + rows_v[slot, l, pl.ds(16 * q, 16)] for q in range(QV)
            )
        zero = jnp.zeros((16,), jnp.float32)
        acc = lax.fori_loop(0, L, inner, (zero,) * QV)
        scale = jnp.float32(1.0 / L)
        for q in range(QV):
            out_v[b // 2, pl.ds((b % 2) * EMB + 16 * q, 16)] = acc[q] * scale

    def outer(g, carry):
        for slot in range(2):
            b = g * 2 + slot
            wait(slot)
            accum(slot, b)
            nb = b + 2

            @pl.when(nb < BPW)
            def _():
                start(nb, slot)
        return carry

    lax.fori_loop(0, BPW // 2, outer, 0)

    pltpu.sync_copy(out_v, out_hbm.at[pl.ds(wid * (BPW // 2), BPW // 2)])


_mesh = plsc.VectorSubcoreMesh(core_axis_name="c", subcore_axis_name="s")

_sc_call = pl.kernel(
    _body,
    mesh=_mesh,
    out_type=jax.ShapeDtypeStruct((B // 2, WPAD), jnp.float32),
    scratch_types=[
        pltpu.VMEM((BPW * L,), jnp.int32),
        pltpu.VMEM((2, L, WPAD), jnp.float32),
        pltpu.VMEM((BPW // 2, WPAD), jnp.float32),
        pltpu.SemaphoreType.DMA,
        pltpu.SemaphoreType.DMA,
    ],
    compiler_params=pltpu.CompilerParams(use_tc_tiling_on_sc=True),
)


@jax.jit
def _run(x, w):
    w128 = jnp.pad(w, ((0, 0), (0, WPAD - EMB)))
    out2 = _sc_call(x.reshape(B * L), w128)
    return out2.reshape(B, EMB)


def kernel(x, sizes, emb_weight):
    del sizes
    return _run(x, emb_weight)
